# baseline (speedup 1.0000x reference)
import functools

import jax
import jax.numpy as jnp
from jax import lax
from jax.experimental import pallas as pl
from jax.experimental.pallas import tpu as pltpu
from jax.experimental.pallas import tpu_sc as plsc

_REG = 0.001
_B = 16384
_D = 64
_NC = 2
_NS = 16
_L = 16
_NW = _NC * _NS
_BPW = _B // _NW
_CHUNK = 128
_NCHUNK = _BPW // _CHUNK


def _sc_body(user_hbm, ii_hbm, ij_hbm, eu2_hbm, ei2_hbm,
             pi_hbm, pj_hbm, reg_hbm,
             uidx_v, iidx_v, jidx_v, upr_v, ipr_v, jpr_v,
             u2_v, vi2_v, vj2_v, pi_v, pj_v, reg_v,
             sem):
    c = lax.axis_index("c")
    s = lax.axis_index("s")
    wid = s * _NC + c
    base = wid * _BPW

    pltpu.sync_copy(user_hbm.at[pl.ds(base, _BPW)], uidx_v)
    pltpu.sync_copy(ii_hbm.at[pl.ds(base, _BPW)], iidx_v)
    pltpu.sync_copy(ij_hbm.at[pl.ds(base, _BPW)], jidx_v)

    def halve(i, _):
        sl = pl.ds(i * _L, _L)
        upr_v[sl] = lax.shift_right_logical(uidx_v[sl], 1)
        ipr_v[sl] = lax.shift_right_logical(iidx_v[sl], 1)
        jpr_v[sl] = lax.shift_right_logical(jidx_v[sl], 1)
        return 0
    lax.fori_loop(0, _BPW // _L, halve, 0)

    lane = lax.iota(jnp.int32, _L)
    reg_acc0 = jnp.zeros((_L,), jnp.float32)

    def make_group(chunk_off):
        def group(g, reg_acc):
            goff = chunk_off + g * _L
            pu = lax.bitwise_and(uidx_v[pl.ds(goff, _L)], 1) * _D
            pi_ = lax.bitwise_and(iidx_v[pl.ds(goff, _L)], 1) * _D
            pj_ = lax.bitwise_and(jidx_v[pl.ds(goff, _L)], 1) * _D
            acc_i = jnp.zeros((_L,), jnp.float32)
            acc_j = jnp.zeros((_L,), jnp.float32)
            for l in range(_L):
                r = g * _L + l
                hu = pu[l]
                hi = pi_[l]
                hj = pj_[l]
                ss_i = jnp.zeros((_L,), jnp.float32)
                ss_j = jnp.zeros((_L,), jnp.float32)
                for cc in range(_D // _L):
                    uu = u2_v[r, pl.ds(hu + cc * _L, _L)]
                    vv = vi2_v[r, pl.ds(hi + cc * _L, _L)]
                    ww = vj2_v[r, pl.ds(hj + cc * _L, _L)]
                    ss_i = ss_i + uu * vv
                    ss_j = ss_j + uu * ww
                    reg_acc = reg_acc + uu * uu + vv * vv + ww * ww
                acc_i = jnp.where(lane == l, jnp.sum(ss_i), acc_i)
                acc_j = jnp.where(lane == l, jnp.sum(ss_j), acc_j)
            row = pl.ds(chunk_off + g * _L, _L)
            pi_v[row] = acc_i
            pj_v[row] = acc_j
            return reg_acc
        return group

    reg_acc = reg_acc0
    for k in range(_NCHUNK):
        sl = pl.ds(k * _CHUNK, _CHUNK)
        cp1 = pltpu.async_copy(eu2_hbm.at[upr_v.at[sl]], u2_v, sem)
        cp2 = pltpu.async_copy(ei2_hbm.at[ipr_v.at[sl]], vi2_v, sem)
        cp3 = pltpu.async_copy(ei2_hbm.at[jpr_v.at[sl]], vj2_v, sem)
        cp1.wait()
        cp2.wait()
        cp3.wait()
        reg_acc = lax.fori_loop(0, _CHUNK // _L, make_group(k * _CHUNK),
                                reg_acc)

    reg_v[...] = reg_acc
    pltpu.sync_copy(pi_v, pi_hbm.at[pl.ds(base, _BPW)])
    pltpu.sync_copy(pj_v, pj_hbm.at[pl.ds(base, _BPW)])
    pltpu.sync_copy(reg_v, reg_hbm.at[pl.ds(wid * _L, _L)])


_sc_call = functools.partial(
    pl.kernel,
    out_type=(
        jax.ShapeDtypeStruct((_B,), jnp.float32),
        jax.ShapeDtypeStruct((_B,), jnp.float32),
        jax.ShapeDtypeStruct((_NW * _L,), jnp.float32),
    ),
    mesh=plsc.VectorSubcoreMesh(
        core_axis_name="c", subcore_axis_name="s",
        num_cores=_NC, num_subcores=_NS),
    compiler_params=pltpu.CompilerParams(
        needs_layout_passes=False, use_tc_tiling_on_sc=True),
    scratch_types=[
        pltpu.VMEM((_BPW,), jnp.int32),
        pltpu.VMEM((_BPW,), jnp.int32),
        pltpu.VMEM((_BPW,), jnp.int32),
        pltpu.VMEM((_BPW,), jnp.int32),
        pltpu.VMEM((_BPW,), jnp.int32),
        pltpu.VMEM((_BPW,), jnp.int32),
        pltpu.VMEM((_CHUNK, 2 * _D), jnp.float32),
        pltpu.VMEM((_CHUNK, 2 * _D), jnp.float32),
        pltpu.VMEM((_CHUNK, 2 * _D), jnp.float32),
        pltpu.VMEM((_BPW,), jnp.float32),
        pltpu.VMEM((_BPW,), jnp.float32),
        pltpu.VMEM((_L,), jnp.float32),
        pltpu.SemaphoreType.DMA,
    ],
)(_sc_body)


def _pair_rows(table):
    n = table.shape[0]
    pad = (-n) % 2
    if pad:
        table = jnp.concatenate(
            [table, jnp.zeros((pad, _D), table.dtype)], axis=0)
    return table.reshape((n + pad) // 2, 2 * _D)


def _loss_body(pi_ref, pj_ref, reg_ref, out_ref):
    x = pi_ref[...] - pj_ref[...]
    ls = jnp.minimum(x, 0.0) - jnp.log(1.0 + jnp.exp(-jnp.abs(x)))
    out_ref[0, 0] = _REG * jnp.sum(reg_ref[...]) - jnp.sum(ls)


_loss_call = pl.pallas_call(
    _loss_body,
    out_shape=jax.ShapeDtypeStruct((1, 1), jnp.float32),
    out_specs=pl.BlockSpec(memory_space=pltpu.SMEM),
)


def kernel(user, item_i, item_j, embed_user, embed_item):
    eu2 = _pair_rows(embed_user)
    ei2 = _pair_rows(embed_item)
    pi, pj, regp = _sc_call(user, item_i, item_j, eu2, ei2)
    loss = _loss_call(pi.reshape(_B // 128, 128),
                      pj.reshape(_B // 128, 128),
                      regp.reshape(_NW * _L // 128, 128))[0, 0]
    return (pi, pj, loss)


# speedup vs baseline: 1.3387x; 1.3387x over previous
import functools

import jax
import jax.numpy as jnp
from jax import lax
from jax.experimental import pallas as pl
from jax.experimental.pallas import tpu as pltpu
from jax.experimental.pallas import tpu_sc as plsc

_REG = 0.001
_B = 16384
_D = 64
_NC = 2
_NS = 16
_L = 16
_NW = _NC * _NS
_BPW = _B // _NW
_CHUNK = 128
_NCHUNK = _BPW // _CHUNK


def _sc_body(user_hbm, ii_hbm, ij_hbm, eu2_hbm, ei2_hbm,
             pi_hbm, pj_hbm, reg_hbm,
             uidx_v, iidx_v, jidx_v, upr_v, ipr_v, jpr_v,
             u2_v, vi2_v, vj2_v, pi_v, pj_v, reg_v,
             sem):
    c = lax.axis_index("c")
    s = lax.axis_index("s")
    wid = s * _NC + c
    base = wid * _BPW

    pltpu.sync_copy(user_hbm.at[pl.ds(base, _BPW)], uidx_v)
    pltpu.sync_copy(ii_hbm.at[pl.ds(base, _BPW)], iidx_v)
    pltpu.sync_copy(ij_hbm.at[pl.ds(base, _BPW)], jidx_v)

    def pair_row(v):
        return lax.bitwise_or(
            lax.shift_left(lax.shift_right_logical(v, 4), 3),
            lax.bitwise_and(v, 7))

    def halve(i, _):
        sl = pl.ds(i * _L, _L)
        upr_v[sl] = pair_row(uidx_v[sl])
        ipr_v[sl] = pair_row(iidx_v[sl])
        jpr_v[sl] = pair_row(jidx_v[sl])
        return 0
    lax.fori_loop(0, _BPW // _L, halve, 0)

    lane = lax.iota(jnp.int32, _L)
    reg_acc0 = jnp.zeros((_L,), jnp.float32)

    def make_group(chunk_off):
        def group(g, reg_acc):
            goff = chunk_off + g * _L
            def half_off(v):
                return lax.bitwise_and(lax.shift_right_logical(v, 3), 1) * _D
            pu = half_off(uidx_v[pl.ds(goff, _L)])
            pi_ = half_off(iidx_v[pl.ds(goff, _L)])
            pj_ = half_off(jidx_v[pl.ds(goff, _L)])
            acc_i = jnp.zeros((_L,), jnp.float32)
            acc_j = jnp.zeros((_L,), jnp.float32)
            for l in range(_L):
                r = g * _L + l
                hu = pu[l]
                hi = pi_[l]
                hj = pj_[l]
                ss_i = jnp.zeros((_L,), jnp.float32)
                ss_j = jnp.zeros((_L,), jnp.float32)
                for cc in range(_D // _L):
                    uu = u2_v[r, pl.ds(hu + cc * _L, _L)]
                    vv = vi2_v[r, pl.ds(hi + cc * _L, _L)]
                    ww = vj2_v[r, pl.ds(hj + cc * _L, _L)]
                    ss_i = ss_i + uu * vv
                    ss_j = ss_j + uu * ww
                    reg_acc = reg_acc + uu * uu + vv * vv + ww * ww
                acc_i = jnp.where(lane == l, jnp.sum(ss_i), acc_i)
                acc_j = jnp.where(lane == l, jnp.sum(ss_j), acc_j)
            row = pl.ds(chunk_off + g * _L, _L)
            pi_v[row] = acc_i
            pj_v[row] = acc_j
            return reg_acc
        return group

    reg_acc = reg_acc0
    for k in range(_NCHUNK):
        sl = pl.ds(k * _CHUNK, _CHUNK)
        cp1 = pltpu.async_copy(eu2_hbm.at[upr_v.at[sl]], u2_v, sem)
        cp2 = pltpu.async_copy(ei2_hbm.at[ipr_v.at[sl]], vi2_v, sem)
        cp3 = pltpu.async_copy(ei2_hbm.at[jpr_v.at[sl]], vj2_v, sem)
        cp1.wait()
        cp2.wait()
        cp3.wait()
        reg_acc = lax.fori_loop(0, _CHUNK // _L, make_group(k * _CHUNK),
                                reg_acc)

    reg_v[...] = reg_acc
    pltpu.sync_copy(pi_v, pi_hbm.at[pl.ds(base, _BPW)])
    pltpu.sync_copy(pj_v, pj_hbm.at[pl.ds(base, _BPW)])
    pltpu.sync_copy(reg_v, reg_hbm.at[pl.ds(wid * _L, _L)])


_sc_call = functools.partial(
    pl.kernel,
    out_type=(
        jax.ShapeDtypeStruct((_B,), jnp.float32),
        jax.ShapeDtypeStruct((_B,), jnp.float32),
        jax.ShapeDtypeStruct((_NW * _L,), jnp.float32),
    ),
    mesh=plsc.VectorSubcoreMesh(
        core_axis_name="c", subcore_axis_name="s",
        num_cores=_NC, num_subcores=_NS),
    compiler_params=pltpu.CompilerParams(
        needs_layout_passes=False, use_tc_tiling_on_sc=True),
    scratch_types=[
        pltpu.VMEM((_BPW,), jnp.int32),
        pltpu.VMEM((_BPW,), jnp.int32),
        pltpu.VMEM((_BPW,), jnp.int32),
        pltpu.VMEM((_BPW,), jnp.int32),
        pltpu.VMEM((_BPW,), jnp.int32),
        pltpu.VMEM((_BPW,), jnp.int32),
        pltpu.VMEM((_CHUNK, 2 * _D), jnp.float32),
        pltpu.VMEM((_CHUNK, 2 * _D), jnp.float32),
        pltpu.VMEM((_CHUNK, 2 * _D), jnp.float32),
        pltpu.VMEM((_BPW,), jnp.float32),
        pltpu.VMEM((_BPW,), jnp.float32),
        pltpu.VMEM((_L,), jnp.float32),
        pltpu.SemaphoreType.DMA,
    ],
)(_sc_body)


def _pair_body(t_ref, o_ref):
    x = t_ref[...]
    q = x.shape[0] // 16
    x4 = x.reshape(q, 2, 8, _D)
    out = jnp.concatenate([x4[:, 0], x4[:, 1]], axis=-1)
    o_ref[...] = out.reshape(q * 8, 2 * _D)


def _pair_rows(table):
    n = table.shape[0]
    m = 8 * ((n + 15) // 16)
    blk = 4096
    grid = (m + blk - 1) // blk
    return pl.pallas_call(
        _pair_body,
        grid=(grid,),
        in_specs=[pl.BlockSpec((2 * blk, _D), lambda i: (i, 0))],
        out_specs=pl.BlockSpec((blk, 2 * _D), lambda i: (i, 0)),
        out_shape=jax.ShapeDtypeStruct((m, 2 * _D), table.dtype),
    )(table)


def _loss_body(pi_ref, pj_ref, reg_ref, out_ref):
    x = pi_ref[...] - pj_ref[...]
    ls = jnp.minimum(x, 0.0) - jnp.log(1.0 + jnp.exp(-jnp.abs(x)))
    out_ref[0, 0] = _REG * jnp.sum(reg_ref[...]) - jnp.sum(ls)


_loss_call = pl.pallas_call(
    _loss_body,
    out_shape=jax.ShapeDtypeStruct((1, 1), jnp.float32),
    out_specs=pl.BlockSpec(memory_space=pltpu.SMEM),
)


def kernel(user, item_i, item_j, embed_user, embed_item):
    eu2 = _pair_rows(embed_user)
    ei2 = _pair_rows(embed_item)
    pi, pj, regp = _sc_call(user, item_i, item_j, eu2, ei2)
    loss = _loss_call(pi.reshape(_B // 128, 128),
                      pj.reshape(_B // 128, 128),
                      regp.reshape(_NW * _L // 128, 128))[0, 0]
    return (pi, pj, loss)


# speedup vs baseline: 2.0982x; 1.5673x over previous
import functools

import jax
import jax.numpy as jnp
from jax import lax
from jax.experimental import pallas as pl
from jax.experimental.pallas import tpu as pltpu
from jax.experimental.pallas import tpu_sc as plsc

_REG = 0.001
_B = 16384
_D = 64
_NC = 2
_NS = 16
_L = 16
_NW = _NC * _NS
_BPW = _B // _NW
_CHUNK = 128
_NCHUNK = _BPW // _CHUNK


def _sc_body(user_hbm, ii_hbm, ij_hbm, eu2_hbm, ei2_hbm,
             pi_hbm, pj_hbm, reg_hbm,
             uidx_v, iidx_v, jidx_v, upr_v, ipr_v, jpr_v,
             u2_v, vi2_v, vj2_v, pi_v, pj_v, reg_v,
             sem):
    c = lax.axis_index("c")
    s = lax.axis_index("s")
    wid = s * _NC + c
    base = wid * _BPW

    pltpu.sync_copy(user_hbm.at[pl.ds(base, _BPW)], uidx_v)
    pltpu.sync_copy(ii_hbm.at[pl.ds(base, _BPW)], iidx_v)
    pltpu.sync_copy(ij_hbm.at[pl.ds(base, _BPW)], jidx_v)

    def pair_row(v):
        return lax.bitwise_or(
            lax.shift_left(lax.shift_right_logical(v, 4), 3),
            lax.bitwise_and(v, 7))

    def halve(i, _):
        sl = pl.ds(i * _L, _L)
        upr_v[sl] = pair_row(uidx_v[sl])
        ipr_v[sl] = pair_row(iidx_v[sl])
        jpr_v[sl] = pair_row(jidx_v[sl])
        return 0
    lax.fori_loop(0, _BPW // _L, halve, 0)

    lane = lax.iota(jnp.int32, _L)
    reg_acc0 = jnp.zeros((_L,), jnp.float32)

    def make_group(chunk_off):
        def group(g, reg_acc):
            goff = chunk_off + g * _L
            def half_off(v):
                return lax.bitwise_and(lax.shift_right_logical(v, 3), 1) * _D
            pu = half_off(uidx_v[pl.ds(goff, _L)])
            pi_ = half_off(iidx_v[pl.ds(goff, _L)])
            pj_ = half_off(jidx_v[pl.ds(goff, _L)])
            acc_i = jnp.zeros((_L,), jnp.float32)
            acc_j = jnp.zeros((_L,), jnp.float32)
            for l in range(_L):
                r = g * _L + l
                hu = pu[l]
                hi = pi_[l]
                hj = pj_[l]
                ss_i = jnp.zeros((_L,), jnp.float32)
                ss_j = jnp.zeros((_L,), jnp.float32)
                for cc in range(_D // _L):
                    uu = u2_v[r, pl.ds(hu + cc * _L, _L)]
                    vv = vi2_v[r, pl.ds(hi + cc * _L, _L)]
                    ww = vj2_v[r, pl.ds(hj + cc * _L, _L)]
                    ss_i = ss_i + uu * vv
                    ss_j = ss_j + uu * ww
                    reg_acc = reg_acc + uu * uu + vv * vv + ww * ww
                acc_i = jnp.where(lane == l, jnp.sum(ss_i), acc_i)
                acc_j = jnp.where(lane == l, jnp.sum(ss_j), acc_j)
            row = pl.ds(chunk_off + g * _L, _L)
            pi_v[row] = acc_i
            pj_v[row] = acc_j
            return reg_acc
        return group

    reg_acc = reg_acc0
    for k in range(_NCHUNK):
        sl = pl.ds(k * _CHUNK, _CHUNK)
        cp1 = pltpu.async_copy(eu2_hbm.at[upr_v.at[sl]], u2_v, sem)
        cp2 = pltpu.async_copy(ei2_hbm.at[ipr_v.at[sl]], vi2_v, sem)
        cp3 = pltpu.async_copy(ei2_hbm.at[jpr_v.at[sl]], vj2_v, sem)
        cp1.wait()
        cp2.wait()
        cp3.wait()
        reg_acc = lax.fori_loop(0, _CHUNK // _L, make_group(k * _CHUNK),
                                reg_acc)

    reg_v[...] = reg_acc
    pltpu.sync_copy(pi_v, pi_hbm.at[pl.ds(base, _BPW)])
    pltpu.sync_copy(pj_v, pj_hbm.at[pl.ds(base, _BPW)])
    pltpu.sync_copy(reg_v, reg_hbm.at[pl.ds(wid * _L, _L)])


_sc_call = functools.partial(
    pl.kernel,
    out_type=(
        jax.ShapeDtypeStruct((_B,), jnp.float32),
        jax.ShapeDtypeStruct((_B,), jnp.float32),
        jax.ShapeDtypeStruct((_NW * _L,), jnp.float32),
    ),
    mesh=plsc.VectorSubcoreMesh(
        core_axis_name="c", subcore_axis_name="s",
        num_cores=_NC, num_subcores=_NS),
    compiler_params=pltpu.CompilerParams(
        needs_layout_passes=False, use_tc_tiling_on_sc=True),
    scratch_types=[
        pltpu.VMEM((_BPW,), jnp.int32),
        pltpu.VMEM((_BPW,), jnp.int32),
        pltpu.VMEM((_BPW,), jnp.int32),
        pltpu.VMEM((_BPW,), jnp.int32),
        pltpu.VMEM((_BPW,), jnp.int32),
        pltpu.VMEM((_BPW,), jnp.int32),
        pltpu.VMEM((_CHUNK, 2 * _D), jnp.float32),
        pltpu.VMEM((_CHUNK, 2 * _D), jnp.float32),
        pltpu.VMEM((_CHUNK, 2 * _D), jnp.float32),
        pltpu.VMEM((_BPW,), jnp.float32),
        pltpu.VMEM((_BPW,), jnp.float32),
        pltpu.VMEM((_L,), jnp.float32),
        pltpu.SemaphoreType.DMA,
    ],
)(_sc_body)


def _pair_body(t_ref, o_ref):
    x = jnp.swapaxes(t_ref[...], 0, 1)
    q = x.shape[0] // 16
    x4 = x.reshape(q, 2, 8, _D)
    out = jnp.concatenate([x4[:, 0], x4[:, 1]], axis=-1)
    o_ref[...] = out.reshape(q * 8, 2 * _D)


def _pair_rows(table):
    n = table.shape[0]
    m = 8 * ((n + 15) // 16)
    blk = 4096
    grid = (m + blk - 1) // blk
    return pl.pallas_call(
        _pair_body,
        grid=(grid,),
        in_specs=[pl.BlockSpec((_D, 2 * blk), lambda i: (0, i))],
        out_specs=pl.BlockSpec((blk, 2 * _D), lambda i: (i, 0)),
        out_shape=jax.ShapeDtypeStruct((m, 2 * _D), table.dtype),
    )(table.T)


def _loss_body(pi_ref, pj_ref, reg_ref, out_ref):
    x = pi_ref[...] - pj_ref[...]
    ls = jnp.minimum(x, 0.0) - jnp.log(1.0 + jnp.exp(-jnp.abs(x)))
    out_ref[0, 0] = _REG * jnp.sum(reg_ref[...]) - jnp.sum(ls)


_loss_call = pl.pallas_call(
    _loss_body,
    out_shape=jax.ShapeDtypeStruct((1, 1), jnp.float32),
    out_specs=pl.BlockSpec(memory_space=pltpu.SMEM),
)


def kernel(user, item_i, item_j, embed_user, embed_item):
    eu2 = _pair_rows(embed_user)
    ei2 = _pair_rows(embed_item)
    pi, pj, regp = _sc_call(user, item_i, item_j, eu2, ei2)
    loss = _loss_call(pi.reshape(_B // 128, 128),
                      pj.reshape(_B // 128, 128),
                      regp.reshape(_NW * _L // 128, 128))[0, 0]
    return (pi, pj, loss)


# speedup vs baseline: 2.2802x; 1.0867x over previous
import functools

import jax
import jax.numpy as jnp
from jax import lax
from jax.experimental import pallas as pl
from jax.experimental.pallas import tpu as pltpu
from jax.experimental.pallas import tpu_sc as plsc

_REG = 0.001
_B = 16384
_D = 64
_NC = 2
_NS = 16
_L = 16
_NW = _NC * _NS
_BPW = _B // _NW
_CHUNK = 128
_NCHUNK = _BPW // _CHUNK


def _sc_body(user_hbm, ii_hbm, ij_hbm, eu2_hbm, ei2_hbm,
             pi_hbm, pj_hbm, reg_hbm,
             uidx_v, iidx_v, jidx_v, upr_v, ipr_v, jpr_v,
             u2a, vi2a, vj2a, u2b, vi2b, vj2b, pi_v, pj_v, reg_v,
             sem_a, sem_b):
    c = lax.axis_index("c")
    s = lax.axis_index("s")
    wid = s * _NC + c
    base = wid * _BPW

    pltpu.sync_copy(user_hbm.at[pl.ds(base, _BPW)], uidx_v)
    pltpu.sync_copy(ii_hbm.at[pl.ds(base, _BPW)], iidx_v)
    pltpu.sync_copy(ij_hbm.at[pl.ds(base, _BPW)], jidx_v)

    def pair_row(v):
        return lax.bitwise_or(
            lax.shift_left(lax.shift_right_logical(v, 4), 3),
            lax.bitwise_and(v, 7))

    def halve(i, _):
        sl = pl.ds(i * _L, _L)
        upr_v[sl] = pair_row(uidx_v[sl])
        ipr_v[sl] = pair_row(iidx_v[sl])
        jpr_v[sl] = pair_row(jidx_v[sl])
        return 0
    lax.fori_loop(0, _BPW // _L, halve, 0)

    lane = lax.iota(jnp.int32, _L)
    reg_acc0 = jnp.zeros((_L,), jnp.float32)

    def make_group(chunk_off, u2_v, vi2_v, vj2_v):
        def group(g, reg_acc):
            goff = chunk_off + g * _L
            def half_off(v):
                return lax.bitwise_and(lax.shift_right_logical(v, 3), 1) * _D
            pu = half_off(uidx_v[pl.ds(goff, _L)])
            pi_ = half_off(iidx_v[pl.ds(goff, _L)])
            pj_ = half_off(jidx_v[pl.ds(goff, _L)])
            acc_i = jnp.zeros((_L,), jnp.float32)
            acc_j = jnp.zeros((_L,), jnp.float32)
            for l in range(_L):
                r = g * _L + l
                hu = pu[l]
                hi = pi_[l]
                hj = pj_[l]
                ss_i = jnp.zeros((_L,), jnp.float32)
                ss_j = jnp.zeros((_L,), jnp.float32)
                for cc in range(_D // _L):
                    uu = u2_v[r, pl.ds(hu + cc * _L, _L)]
                    vv = vi2_v[r, pl.ds(hi + cc * _L, _L)]
                    ww = vj2_v[r, pl.ds(hj + cc * _L, _L)]
                    ss_i = ss_i + uu * vv
                    ss_j = ss_j + uu * ww
                    reg_acc = reg_acc + uu * uu + vv * vv + ww * ww
                acc_i = jnp.where(lane == l, jnp.sum(ss_i), acc_i)
                acc_j = jnp.where(lane == l, jnp.sum(ss_j), acc_j)
            row = pl.ds(chunk_off + g * _L, _L)
            pi_v[row] = acc_i
            pj_v[row] = acc_j
            return reg_acc
        return group

    slots = ((u2a, vi2a, vj2a, sem_a), (u2b, vi2b, vj2b, sem_b))

    def fire(k, slot):
        u2_v, vi2_v, vj2_v, sem = slot
        sl = pl.ds(k * _CHUNK, _CHUNK)
        return (pltpu.async_copy(eu2_hbm.at[upr_v.at[sl]], u2_v, sem),
                pltpu.async_copy(ei2_hbm.at[ipr_v.at[sl]], vi2_v, sem),
                pltpu.async_copy(ei2_hbm.at[jpr_v.at[sl]], vj2_v, sem))

    reg_acc = reg_acc0
    inflight = fire(0, slots[0])
    for k in range(_NCHUNK):
        slot = slots[k % 2]
        for cp in inflight:
            cp.wait()
        if k + 1 < _NCHUNK:
            inflight = fire(k + 1, slots[(k + 1) % 2])
        reg_acc = lax.fori_loop(
            0, _CHUNK // _L, make_group(k * _CHUNK, *slot[:3]), reg_acc)

    reg_v[...] = reg_acc
    pltpu.sync_copy(pi_v, pi_hbm.at[pl.ds(base, _BPW)])
    pltpu.sync_copy(pj_v, pj_hbm.at[pl.ds(base, _BPW)])
    pltpu.sync_copy(reg_v, reg_hbm.at[pl.ds(wid * _L, _L)])


_sc_call = functools.partial(
    pl.kernel,
    out_type=(
        jax.ShapeDtypeStruct((_B,), jnp.float32),
        jax.ShapeDtypeStruct((_B,), jnp.float32),
        jax.ShapeDtypeStruct((_NW * _L,), jnp.float32),
    ),
    mesh=plsc.VectorSubcoreMesh(
        core_axis_name="c", subcore_axis_name="s",
        num_cores=_NC, num_subcores=_NS),
    compiler_params=pltpu.CompilerParams(
        needs_layout_passes=False, use_tc_tiling_on_sc=True),
    scratch_types=[
        pltpu.VMEM((_BPW,), jnp.int32),
        pltpu.VMEM((_BPW,), jnp.int32),
        pltpu.VMEM((_BPW,), jnp.int32),
        pltpu.VMEM((_BPW,), jnp.int32),
        pltpu.VMEM((_BPW,), jnp.int32),
        pltpu.VMEM((_BPW,), jnp.int32),
        pltpu.VMEM((_CHUNK, 2 * _D), jnp.float32),
        pltpu.VMEM((_CHUNK, 2 * _D), jnp.float32),
        pltpu.VMEM((_CHUNK, 2 * _D), jnp.float32),
        pltpu.VMEM((_CHUNK, 2 * _D), jnp.float32),
        pltpu.VMEM((_CHUNK, 2 * _D), jnp.float32),
        pltpu.VMEM((_CHUNK, 2 * _D), jnp.float32),
        pltpu.VMEM((_BPW,), jnp.float32),
        pltpu.VMEM((_BPW,), jnp.float32),
        pltpu.VMEM((_L,), jnp.float32),
        pltpu.SemaphoreType.DMA,
        pltpu.SemaphoreType.DMA,
    ],
)(_sc_body)


def _pair_body(t_ref, o_ref):
    x = jnp.swapaxes(t_ref[...], 0, 1)
    q = x.shape[0] // 16
    x4 = x.reshape(q, 2, 8, _D)
    out = jnp.concatenate([x4[:, 0], x4[:, 1]], axis=-1)
    o_ref[...] = out.reshape(q * 8, 2 * _D)


def _pair_rows(table):
    n = table.shape[0]
    m = 8 * ((n + 15) // 16)
    blk = 4096
    grid = (m + blk - 1) // blk
    return pl.pallas_call(
        _pair_body,
        grid=(grid,),
        in_specs=[pl.BlockSpec((_D, 2 * blk), lambda i: (0, i))],
        out_specs=pl.BlockSpec((blk, 2 * _D), lambda i: (i, 0)),
        out_shape=jax.ShapeDtypeStruct((m, 2 * _D), table.dtype),
    )(table.T)


def _loss_body(pi_ref, pj_ref, reg_ref, out_ref):
    x = pi_ref[...] - pj_ref[...]
    ls = jnp.minimum(x, 0.0) - jnp.log(1.0 + jnp.exp(-jnp.abs(x)))
    out_ref[0, 0] = _REG * jnp.sum(reg_ref[...]) - jnp.sum(ls)


_loss_call = pl.pallas_call(
    _loss_body,
    out_shape=jax.ShapeDtypeStruct((1, 1), jnp.float32),
    out_specs=pl.BlockSpec(memory_space=pltpu.SMEM),
)


def kernel(user, item_i, item_j, embed_user, embed_item):
    eu2 = _pair_rows(embed_user)
    ei2 = _pair_rows(embed_item)
    pi, pj, regp = _sc_call(user, item_i, item_j, eu2, ei2)
    loss = _loss_call(pi.reshape(_B // 128, 128),
                      pj.reshape(_B // 128, 128),
                      regp.reshape(_NW * _L // 128, 128))[0, 0]
    return (pi, pj, loss)


# speedup vs baseline: 2.3165x; 1.0159x over previous
import functools

import jax
import jax.numpy as jnp
from jax import lax
from jax.experimental import pallas as pl
from jax.experimental.pallas import tpu as pltpu
from jax.experimental.pallas import tpu_sc as plsc

_REG = 0.001
_B = 16384
_D = 64
_NC = 2
_NS = 16
_L = 16
_NW = _NC * _NS
_BPW = _B // _NW
_CHUNK = 128
_NCHUNK = _BPW // _CHUNK


def _sc_body(user_hbm, ii_hbm, ij_hbm, eu2_hbm, ei2_hbm,
             pi_hbm, pj_hbm, reg_hbm,
             uidx_v, iidx_v, jidx_v, upr_v, ipr_v, jpr_v,
             u2a, vi2a, vj2a, u2b, vi2b, vj2b, pi_v, pj_v, reg_v,
             sem_a, sem_b):
    c = lax.axis_index("c")
    s = lax.axis_index("s")
    wid = s * _NC + c
    base = wid * _BPW

    pltpu.sync_copy(user_hbm.at[pl.ds(base, _BPW)], uidx_v)
    pltpu.sync_copy(ii_hbm.at[pl.ds(base, _BPW)], iidx_v)
    pltpu.sync_copy(ij_hbm.at[pl.ds(base, _BPW)], jidx_v)

    def pair_row(v):
        return lax.bitwise_or(
            lax.shift_left(lax.shift_right_logical(v, 4), 3),
            lax.bitwise_and(v, 7))

    def halve(i, _):
        sl = pl.ds(i * _L, _L)
        upr_v[sl] = pair_row(uidx_v[sl])
        ipr_v[sl] = pair_row(iidx_v[sl])
        jpr_v[sl] = pair_row(jidx_v[sl])
        return 0
    lax.fori_loop(0, _BPW // _L, halve, 0)

    lane = lax.iota(jnp.int32, _L)
    reg_acc0 = jnp.zeros((_L,), jnp.float32)

    def make_group(chunk_off, u2_v, vi2_v, vj2_v):
        def group(g, reg_acc):
            goff = chunk_off + g * _L
            def half_off(v):
                return lax.bitwise_and(lax.shift_right_logical(v, 3), 1) * _D
            pu = half_off(uidx_v[pl.ds(goff, _L)])
            pi_ = half_off(iidx_v[pl.ds(goff, _L)])
            pj_ = half_off(jidx_v[pl.ds(goff, _L)])
            acc_i = jnp.zeros((_L,), jnp.float32)
            acc_j = jnp.zeros((_L,), jnp.float32)
            for l in range(_L):
                r = g * _L + l
                hu = pu[l]
                hi = pi_[l]
                hj = pj_[l]
                ss_i = jnp.zeros((_L,), jnp.float32)
                ss_j = jnp.zeros((_L,), jnp.float32)
                for cc in range(_D // _L):
                    uu = u2_v[r, pl.ds(hu + cc * _L, _L)]
                    vv = vi2_v[r, pl.ds(hi + cc * _L, _L)]
                    ww = vj2_v[r, pl.ds(hj + cc * _L, _L)]
                    ss_i = ss_i + uu * vv
                    ss_j = ss_j + uu * ww
                    reg_acc = reg_acc + uu * uu + vv * vv + ww * ww
                acc_i = jnp.where(lane == l, jnp.sum(ss_i), acc_i)
                acc_j = jnp.where(lane == l, jnp.sum(ss_j), acc_j)
            row = pl.ds(chunk_off + g * _L, _L)
            pi_v[row] = acc_i
            pj_v[row] = acc_j
            return reg_acc
        return group

    slots = ((u2a, vi2a, vj2a, sem_a), (u2b, vi2b, vj2b, sem_b))

    def fire(k, slot):
        u2_v, vi2_v, vj2_v, sem = slot
        sl = pl.ds(k * _CHUNK, _CHUNK)
        return (pltpu.async_copy(eu2_hbm.at[upr_v.at[sl]], u2_v, sem),
                pltpu.async_copy(ei2_hbm.at[ipr_v.at[sl]], vi2_v, sem),
                pltpu.async_copy(ei2_hbm.at[jpr_v.at[sl]], vj2_v, sem))

    reg_acc = reg_acc0
    inflight = fire(0, slots[0])
    for k in range(_NCHUNK):
        slot = slots[k % 2]
        for cp in inflight:
            cp.wait()
        if k + 1 < _NCHUNK:
            inflight = fire(k + 1, slots[(k + 1) % 2])
        reg_acc = lax.fori_loop(
            0, _CHUNK // _L, make_group(k * _CHUNK, *slot[:3]), reg_acc)

    reg_v[...] = reg_acc
    pltpu.sync_copy(pi_v, pi_hbm.at[pl.ds(base, _BPW)])
    pltpu.sync_copy(pj_v, pj_hbm.at[pl.ds(base, _BPW)])
    pltpu.sync_copy(reg_v, reg_hbm.at[pl.ds(wid * _L, _L)])


_sc_call = functools.partial(
    pl.kernel,
    out_type=(
        jax.ShapeDtypeStruct((_B,), jnp.float32),
        jax.ShapeDtypeStruct((_B,), jnp.float32),
        jax.ShapeDtypeStruct((_NW * _L,), jnp.float32),
    ),
    mesh=plsc.VectorSubcoreMesh(
        core_axis_name="c", subcore_axis_name="s",
        num_cores=_NC, num_subcores=_NS),
    compiler_params=pltpu.CompilerParams(
        needs_layout_passes=False, use_tc_tiling_on_sc=True),
    scratch_types=[
        pltpu.VMEM((_BPW,), jnp.int32),
        pltpu.VMEM((_BPW,), jnp.int32),
        pltpu.VMEM((_BPW,), jnp.int32),
        pltpu.VMEM((_BPW,), jnp.int32),
        pltpu.VMEM((_BPW,), jnp.int32),
        pltpu.VMEM((_BPW,), jnp.int32),
        pltpu.VMEM((_CHUNK, 2 * _D), jnp.float32),
        pltpu.VMEM((_CHUNK, 2 * _D), jnp.float32),
        pltpu.VMEM((_CHUNK, 2 * _D), jnp.float32),
        pltpu.VMEM((_CHUNK, 2 * _D), jnp.float32),
        pltpu.VMEM((_CHUNK, 2 * _D), jnp.float32),
        pltpu.VMEM((_CHUNK, 2 * _D), jnp.float32),
        pltpu.VMEM((_BPW,), jnp.float32),
        pltpu.VMEM((_BPW,), jnp.float32),
        pltpu.VMEM((_L,), jnp.float32),
        pltpu.SemaphoreType.DMA,
        pltpu.SemaphoreType.DMA,
    ],
)(_sc_body)


def _pair_body(t_ref, o_ref):
    x = jnp.swapaxes(t_ref[...], 0, 1)
    q = x.shape[0] // 16
    x4 = x.reshape(q, 2, 8, _D)
    out = jnp.concatenate([x4[:, 0], x4[:, 1]], axis=-1)
    o_ref[...] = out.reshape(q * 8, 2 * _D)


def _pair_rows(table):
    n = table.shape[0]
    m = 8 * ((n + 15) // 16)
    blk = 8192
    grid = (m + blk - 1) // blk
    return pl.pallas_call(
        _pair_body,
        grid=(grid,),
        in_specs=[pl.BlockSpec((_D, 2 * blk), lambda i: (0, i))],
        out_specs=pl.BlockSpec((blk, 2 * _D), lambda i: (i, 0)),
        out_shape=jax.ShapeDtypeStruct((m, 2 * _D), table.dtype),
    )(table.T)


def _loss_body(pi_ref, pj_ref, reg_ref, out_ref):
    x = pi_ref[...] - pj_ref[...]
    ls = jnp.minimum(x, 0.0) - jnp.log(1.0 + jnp.exp(-jnp.abs(x)))
    out_ref[0, 0] = _REG * jnp.sum(reg_ref[...]) - jnp.sum(ls)


_loss_call = pl.pallas_call(
    _loss_body,
    out_shape=jax.ShapeDtypeStruct((1, 1), jnp.float32),
    out_specs=pl.BlockSpec(memory_space=pltpu.SMEM),
)


def kernel(user, item_i, item_j, embed_user, embed_item):
    eu2 = _pair_rows(embed_user)
    ei2 = _pair_rows(embed_item)
    pi, pj, regp = _sc_call(user, item_i, item_j, eu2, ei2)
    loss = _loss_call(pi.reshape(_B // 128, 128),
                      pj.reshape(_B // 128, 128),
                      regp.reshape(_NW * _L // 128, 128))[0, 0]
    return (pi, pj, loss)
